# baseline (speedup 1.0000x reference)
import functools

import jax
import jax.numpy as jnp
from jax import lax
from jax.experimental import pallas as pl
from jax.experimental.pallas import tpu as pltpu
from jax.experimental.pallas import tpu_sc as plsc

B, S, D = 4096, 50, 100
VOCAB = 100000
NC, NS = 2, 16
NW = NC * NS
RPW = B // NW
C = 8
NCH = RPW // C
CS = C * S
IDXW = 128
NIDX = 4
CSP = NIDX * IDXW
OFFS = (0, 16, 32, 48, 64, 80, 84)


def _build_sc_kernel():
    mesh = plsc.VectorSubcoreMesh(core_axis_name="c", subcore_axis_name="s")

    @functools.partial(
        pl.kernel,
        mesh=mesh,
        out_type=jax.ShapeDtypeStruct((B, D), jnp.float32),
        scratch_types=[
            pltpu.VMEM((CS,), jnp.int32),
            pltpu.VMEM((CS,), jnp.int32),
            pltpu.VMEM((CSP,), jnp.int32),
            pltpu.VMEM((CSP,), jnp.int32),
            pltpu.VMEM((CSP, D), jnp.float32),
            pltpu.VMEM((CSP, D), jnp.float32),
            pltpu.VMEM((C, D), jnp.float32),
            pltpu.SemaphoreType.DMA,
            pltpu.SemaphoreType.DMA,
        ],
        compiler_params=pltpu.CompilerParams(use_tc_tiling_on_sc=False,
                                             needs_layout_passes=False),
    )
    def k(ids_hbm, mask_hbm, table_hbm, out_hbm,
          ids_v, mask_v, idx_a, idx_b, rows_a, rows_b, out_v,
          sem_a, sem_b):
        wid = lax.axis_index("s") * NC + lax.axis_index("c")
        iota = lax.iota(jnp.int32, 16)

        def filler_rows(i):
            zoff = lax.rem(wid * 2048 + i * 16, 65536)
            return zoff + iota

        def load_compact_fire(ch, idx_v, rows_v, sem, live):
            base = (wid * RPW + ch * C) * S
            pltpu.sync_copy(ids_hbm.at[pl.ds(base, CS)], ids_v)
            pltpu.sync_copy(mask_hbm.at[pl.ds(base, CS)], mask_v)

            def pad_body(i, c2):
                idx_v[pl.ds(i * 16, 16)] = filler_rows(i)
                return c2

            lax.fori_loop(0, CSP // 16, pad_body, 0)

            def compact_body(i, koff):
                mi = mask_v[pl.ds(i * 16, 16)]
                v = ids_v[pl.ds(i * 16, 16)]
                cs = lax.cumsum(mi, axis=0)
                pos = koff + cs - 1
                plsc.store_scatter(idx_v, [pos], v, mask=mi > 0)
                return koff + cs[15]

            n = lax.fori_loop(0, CS // 16, compact_body, jnp.int32(0))

            for j in range(NIDX):
                @pl.when(jnp.logical_and(live, n > j * IDXW))
                def _():
                    pltpu.async_copy(
                        table_hbm.at[idx_v.at[pl.ds(j * IDXW, IDXW)]],
                        rows_v.at[pl.ds(j * IDXW, IDXW)],
                        sem)

            return n

        def wait_gathers(idx_v, rows_v, sem, n):
            for j in range(NIDX):
                @pl.when(n > j * IDXW)
                def _():
                    pltpu.make_async_copy(
                        table_hbm.at[idx_v.at[pl.ds(j * IDXW, IDXW)]],
                        rows_v.at[pl.ds(j * IDXW, IDXW)],
                        sem).wait()

        def pool_out(ch, rows_v):
            r0 = wid * RPW + ch * C

            cnts = []
            for b in range(C):
                m0 = mask_v[pl.ds(b * S, 16)]
                m1 = mask_v[pl.ds(b * S + 16, 16)]
                m2 = mask_v[pl.ds(b * S + 32, 16)]
                m3 = mask_v[pl.ds(b * S + 34, 16)]
                tail = jnp.where(iota >= 14, m3, jnp.zeros((16,), jnp.int32))
                cnts.append(jnp.sum(m0 + m1 + m2) + jnp.sum(tail))

            start = jnp.int32(0)
            for b in range(C):
                c_b = cnts[b]
                cv = jnp.full((16,), c_b, jnp.int32).astype(jnp.float32)
                cntf = jnp.maximum(cv, jnp.float32(1e-9))
                rcp = jnp.float32(1.0) / cntf

                def sum_body(r, accs):
                    return tuple(accs[kk] + rows_v[r, pl.ds(OFFS[kk], 16)]
                                 for kk in range(7))

                accs = lax.fori_loop(
                    start, start + c_b, sum_body,
                    tuple(jnp.zeros((16,), jnp.float32) for _ in range(7)))
                start = start + c_b
                for kk in range(7):
                    out_v[b, pl.ds(OFFS[kk], 16)] = accs[kk] * rcp

            pltpu.sync_copy(out_v, out_hbm.at[pl.ds(r0, C)])

        n_a0 = load_compact_fire(0, idx_a, rows_a, sem_a, jnp.bool_(True))

        def pipe_body(g, n_a):
            ch0 = 2 * g
            n_b = load_compact_fire(ch0 + 1, idx_b, rows_b, sem_b,
                                    jnp.bool_(True))
            wait_gathers(idx_a, rows_a, sem_a, n_a)
            pool_out(ch0, rows_a)

            live = g < NCH // 2 - 1
            ch_next = jnp.minimum(ch0 + 2, NCH - 1)
            n_a_next = load_compact_fire(ch_next, idx_a, rows_a, sem_a,
                                         live)
            n_a_next = jnp.where(live, n_a_next, jnp.int32(0))

            wait_gathers(idx_b, rows_b, sem_b, n_b)
            pool_out(ch0 + 1, rows_b)
            return n_a_next

        lax.fori_loop(0, NCH // 2, pipe_body, n_a0)

    return k


_SC_KERNEL = _build_sc_kernel()


def kernel(input_ids, attention_mask, embedding_table):
    ids = input_ids.reshape(-1).astype(jnp.int32)
    msk = attention_mask.reshape(-1).astype(jnp.int32)
    tbl = embedding_table.astype(jnp.float32)
    return _SC_KERNEL(ids, msk, tbl)


# speedup vs baseline: 1.3528x; 1.3528x over previous
import functools

import jax
import jax.numpy as jnp
from jax import lax
from jax.experimental import pallas as pl
from jax.experimental.pallas import tpu as pltpu
from jax.experimental.pallas import tpu_sc as plsc

B, S, D = 4096, 50, 100
PAD_ROW = 100000
NC, NS = 2, 16
NW = NC * NS
RPW = B // NW
C = 16
NCH = RPW // C
CS = C * S
IDXW = 128
NIDX = 7
CSP = NIDX * IDXW

DP = 128
ZBASE = 100002
NZ = 8192
NBLK = 4
def _inv_perm():
    perm = []
    for i in range(NBLK):
        perm += [32 * i + 2 * k for k in range(16)]
        perm += [32 * i + 2 * k + 1 for k in range(16)]
    inv = [0] * (32 * NBLK)
    for pos, col in enumerate(perm):
        inv[col] = pos
    return tuple(inv)

INV_PERM = _inv_perm()


def _build_sc_kernel():
    mesh = plsc.VectorSubcoreMesh(core_axis_name="c", subcore_axis_name="s")

    @functools.partial(
        pl.kernel,
        mesh=mesh,
        out_type=jax.ShapeDtypeStruct((B, DP), jnp.float32),
        scratch_types=[
            pltpu.VMEM((CS,), jnp.int32),
            pltpu.VMEM((CS,), jnp.int32),
            pltpu.VMEM((CS,), jnp.int32),
            pltpu.VMEM((CSP,), jnp.int32),
            pltpu.VMEM((CSP,), jnp.int32),
            pltpu.VMEM((CSP, DP), jnp.bfloat16),
            pltpu.VMEM((CSP, DP), jnp.bfloat16),
            pltpu.VMEM((C, DP), jnp.float32),
            pltpu.SemaphoreType.DMA,
            pltpu.SemaphoreType.DMA,
        ],
        compiler_params=pltpu.CompilerParams(use_tc_tiling_on_sc=False,
                                             needs_layout_passes=False),
    )
    def k(ids_hbm, mask_hbm, table_hbm, out_hbm,
          ids_v, mask_a, mask_b, idx_a, idx_b, rows_a, rows_b, out_v,
          sem_a, sem_b):
        wid = lax.axis_index("s") * NC + lax.axis_index("c")
        iota = lax.iota(jnp.int32, 16)

        def zero_rows(i):
            zoff = wid * (NZ // NW) + lax.rem(i * 16, NZ // NW)
            return ZBASE + zoff + iota

        def load_compact_fire(ch, mask_v, idx_v, rows_v, sem, live):
            base = (wid * RPW + ch * C) * S
            pltpu.sync_copy(ids_hbm.at[pl.ds(base, CS)], ids_v)
            pltpu.sync_copy(mask_hbm.at[pl.ds(base, CS)], mask_v)

            def pad_body(i, c2):
                idx_v[pl.ds(i * 16, 16)] = zero_rows(i)
                return c2

            lax.fori_loop(0, CSP // 16, pad_body, 0)

            def compact_body(i, koff):
                mi = mask_v[pl.ds(i * 16, 16)]
                v = ids_v[pl.ds(i * 16, 16)]
                cs = lax.cumsum(mi, axis=0)
                pos = koff + cs - 1
                plsc.store_scatter(idx_v, [pos], v, mask=mi > 0)
                return koff + cs[15]

            n = lax.fori_loop(0, CS // 16, compact_body, jnp.int32(0))

            for j in range(NIDX):
                @pl.when(jnp.logical_and(live, n > j * IDXW))
                def _():
                    pltpu.async_copy(
                        table_hbm.at[idx_v.at[pl.ds(j * IDXW, IDXW)]],
                        rows_v.at[pl.ds(j * IDXW, IDXW)],
                        sem)

            return n

        def wait_gathers(idx_v, rows_v, sem, n):
            for j in range(NIDX):
                @pl.when(n > j * IDXW)
                def _():
                    pltpu.make_async_copy(
                        table_hbm.at[idx_v.at[pl.ds(j * IDXW, IDXW)]],
                        rows_v.at[pl.ds(j * IDXW, IDXW)],
                        sem).wait()

        def pool_out(ch, mask_v, rows_v):
            r0 = wid * RPW + ch * C

            cnts = []
            for b in range(C):
                m0 = mask_v[pl.ds(b * S, 16)]
                m1 = mask_v[pl.ds(b * S + 16, 16)]
                m2 = mask_v[pl.ds(b * S + 32, 16)]
                m3 = mask_v[pl.ds(b * S + 34, 16)]
                tail = jnp.where(iota >= 14, m3,
                                 jnp.zeros((16,), jnp.int32))
                cnts.append(lax.cumsum(m0 + m1 + m2 + tail, axis=0)[15])

            start = jnp.int32(0)
            for b in range(C):
                c_b = cnts[b]
                cv = jnp.full((16,), c_b, jnp.int32).astype(jnp.float32)
                rcp = jnp.float32(1.0) / jnp.maximum(cv, jnp.float32(1e-9))

                def sum_body(r, accs):
                    new = []
                    for i in range(NBLK):
                        ev, od = plsc.unpack(
                            rows_v[r, pl.ds(32 * i, 32)],
                            format=plsc.PackFormat.INTERLEAVED,
                            preferred_element_type=jnp.float32)
                        new.append(accs[2 * i] + ev)
                        new.append(accs[2 * i + 1] + od)
                    return tuple(new)

                accs = lax.fori_loop(
                    start, start + c_b, sum_body,
                    tuple(jnp.zeros((16,), jnp.float32)
                          for _ in range(2 * NBLK)))
                start = start + c_b
                for i in range(NBLK):
                    out_v[b, pl.ds(32 * i, 16)] = accs[2 * i] * rcp
                    out_v[b, pl.ds(32 * i + 16, 16)] = accs[2 * i + 1] * rcp

            pltpu.sync_copy(out_v, out_hbm.at[pl.ds(r0, C)])

        n_a0 = load_compact_fire(0, mask_a, idx_a, rows_a, sem_a,
                                 jnp.bool_(True))

        def pipe_body(g, n_a):
            ch0 = 2 * g
            n_b = load_compact_fire(ch0 + 1, mask_b, idx_b, rows_b, sem_b,
                                    jnp.bool_(True))
            wait_gathers(idx_a, rows_a, sem_a, n_a)
            pool_out(ch0, mask_a, rows_a)

            live = g < NCH // 2 - 1
            ch_next = jnp.minimum(ch0 + 2, NCH - 1)
            n_a_next = load_compact_fire(ch_next, mask_a, idx_a, rows_a,
                                         sem_a, live)
            n_a_next = jnp.where(live, n_a_next, jnp.int32(0))

            wait_gathers(idx_b, rows_b, sem_b, n_b)
            pool_out(ch0 + 1, mask_b, rows_b)
            return n_a_next

        lax.fori_loop(0, NCH // 2, pipe_body, n_a0)

    return k


_SC_KERNEL = _build_sc_kernel()


def kernel(input_ids, attention_mask, embedding_table):
    ids = input_ids.reshape(-1).astype(jnp.int32)
    msk = attention_mask.astype(jnp.int32)
    tbl = jnp.pad(embedding_table.astype(jnp.bfloat16),
                  ((0, NZ), (0, DP - D)))
    res = _SC_KERNEL(ids, msk.reshape(-1), tbl)
    return res[:, jnp.array(INV_PERM[:D], jnp.int32)]
